# Initial kernel scaffold; baseline (speedup 1.0000x reference)
import jax
import jax.numpy as jnp
from jax.experimental import pallas as pl


def kernel(A, X, agg_weights, agg_bias):
    raise NotImplementedError("write your pallas kernel here")


# speedup vs baseline: 4.0663x; 4.0663x over previous
import functools

import jax
import jax.numpy as jnp
from jax import lax
from jax.experimental import pallas as pl
from jax.experimental.pallas import tpu as pltpu
from jax.experimental.pallas import tpu_sc as plsc

N = 10000
C = 128
K = 25
KP = 32
RB = 8
BIG = 1 << 20

NW = 32
ROWS_PER = 320
NP = NW * ROWS_PER
XPAD_ROWS = N + 8


def _extract_body(a_ref, idx_ref):
    a = a_ref[0]
    iota = lax.broadcasted_iota(jnp.int32, a.shape, 1)
    code = jnp.where(a != 0.0, iota, BIG)
    cols = []
    for _ in range(K):
        mn = jnp.min(code, axis=1, keepdims=True)
        cols.append(mn)
        code = jnp.where(code == mn, BIG, code)
    idx = jnp.concatenate(cols, axis=1)
    safe = jnp.where(idx < BIG, idx, N)
    extra = jnp.broadcast_to(safe[:, :1], (RB, KP - K))
    idx_ref[...] = jnp.concatenate([safe, extra], axis=1)


def _extract(A2):
    return pl.pallas_call(
        _extract_body,
        grid=(N // RB,),
        in_specs=[pl.BlockSpec((1, RB, N), lambda i: (0, i, 0))],
        out_specs=pl.BlockSpec((RB, KP), lambda i: (i, 0)),
        out_shape=jax.ShapeDtypeStruct((N, KP), jnp.int32),
    )(A2[None])


def _gather_max_body(xpad_hbm, idx_hbm, out_hbm, idx_v, gbuf, agg_v, sem):
    wid = lax.axis_index("s") * 2 + lax.axis_index("c")
    base = wid * ROWS_PER
    pltpu.sync_copy(idx_hbm.at[pl.ds(base, ROWS_PER)], idx_v)

    @pl.loop(0, ROWS_PER)
    def _row(r):
        pltpu.async_copy(xpad_hbm.at[idx_v.at[r]], gbuf, sem).wait()
        for cch in range(C // 16):
            sl = pl.ds(cch * 16, 16)
            acc = gbuf[0, sl]
            for k in range(1, KP):
                acc = jnp.maximum(acc, gbuf[k, sl])
            agg_v[r, sl] = acc

    pltpu.sync_copy(agg_v, out_hbm.at[pl.ds(base, ROWS_PER)])


def _gather_max(xpad, idx_p):
    mesh = plsc.VectorSubcoreMesh(core_axis_name="c", subcore_axis_name="s")
    kfn = functools.partial(
        pl.kernel,
        mesh=mesh,
        out_type=jax.ShapeDtypeStruct((NP, C), jnp.float32),
        scratch_types=[
            pltpu.VMEM((ROWS_PER, KP), jnp.int32),
            pltpu.VMEM((KP, C), jnp.float32),
            pltpu.VMEM((ROWS_PER, C), jnp.float32),
            pltpu.SemaphoreType.DMA,
        ],
    )(_gather_max_body)
    return kfn(xpad, idx_p)


def _mlp_body(x_ref, a_ref, w1_ref, w2_ref, b_ref, o_ref):
    acc = jnp.dot(x_ref[...], w1_ref[...], preferred_element_type=jnp.float32)
    acc += jnp.dot(a_ref[...], w2_ref[...], preferred_element_type=jnp.float32)
    o_ref[...] = jnp.maximum(acc + b_ref[...], 0.0)


def _mlp(X2, agg, W, b):
    MB = 1000
    return pl.pallas_call(
        _mlp_body,
        grid=(N // MB,),
        in_specs=[
            pl.BlockSpec((MB, C), lambda i: (i, 0)),
            pl.BlockSpec((MB, C), lambda i: (i, 0)),
            pl.BlockSpec((C, C), lambda i: (0, 0)),
            pl.BlockSpec((C, C), lambda i: (0, 0)),
            pl.BlockSpec((1, C), lambda i: (0, 0)),
        ],
        out_specs=pl.BlockSpec((MB, C), lambda i: (i, 0)),
        out_shape=jax.ShapeDtypeStruct((N, C), jnp.float32),
    )(X2, agg, W[:C], W[C:], b[None])


def kernel(A, X, agg_weights, agg_bias):
    A2 = A[0]
    X2 = X[0]
    idx = _extract(A2)
    idx_p = jnp.pad(idx, ((0, NP - N), (0, 0)), constant_values=N)
    xpad = jnp.pad(X2, ((0, XPAD_ROWS - N), (0, 0)))
    agg = _gather_max(xpad, idx_p)[:N]
    out = _mlp(X2, agg, agg_weights, agg_bias)
    return out[None]


# speedup vs baseline: 6.4095x; 1.5762x over previous
import dataclasses
import functools

import jax
import jax.numpy as jnp
from jax import lax
from jax.experimental import pallas as pl
from jax.experimental.pallas import tpu as pltpu
from jax.experimental.pallas import tpu_sc as plsc

N = 10000
C = 128
K = 25
KP = 32

NW = 32
ROWS_PER = 320
NP = NW * ROWS_PER
XPAD_ROWS = N + 8


def _sage_body(a_hbm, xpad_hbm, out_hbm, abuf, idxb, gbuf, aggb, cntb,
               sa0, sa1, sg0, sg1):
    wid = lax.axis_index("s") * 2 + lax.axis_index("c")
    base = wid * ROWS_PER
    iota16 = lax.iota(jnp.int32, 16)
    nfill = jnp.full((16,), N, jnp.int32)
    zeros16 = jnp.zeros((16,), jnp.int32)
    sa = (sa0, sa1)
    sg = (sg0, sg1)

    def a_row(r):
        return jnp.minimum(base + r, N - 1)

    def start_a(r, p):
        pltpu.make_async_copy(a_hbm.at[a_row(r)], abuf.at[p], sa[p]).start()

    def wait_a(p):
        pltpu.make_async_copy(a_hbm.at[0], abuf.at[p], sa[p]).wait()

    def start_g(p):
        pltpu.make_async_copy(xpad_hbm.at[idxb.at[p]], gbuf.at[p],
                              sg[p]).start()

    def wait_g(p):
        pltpu.make_async_copy(xpad_hbm.at[idxb.at[p]], gbuf.at[p],
                              sg[p]).wait()

    def scan_row(p):
        idxb[p, pl.ds(0, 16)] = nfill
        idxb[p, pl.ds(16, 16)] = nfill
        cntb[...] = zeros16

        @pl.loop(0, N, step=16)
        def _(c):
            v = abuf[p, pl.ds(c, 16)]
            m = v != 0.0

            @pl.when(jnp.any(m))
            def _():
                cnt = cntb[...]
                pos = cnt + plsc.cumsum(m.astype(jnp.int32)) - 1
                sm = jnp.logical_and(m, pos < K)
                posc = jnp.minimum(pos, KP - 1)
                plsc.store_scatter(idxb.at[p], [posc], iota16 + c, mask=sm)
                cntb[...] = cnt + plsc.all_reduce_population_count(m)

        idx0 = plsc.load_gather(idxb.at[p], [zeros16])
        hi = idxb[p, pl.ds(16, 16)]
        idxb[p, pl.ds(16, 16)] = jnp.where(iota16 >= K - 16, idx0, hi)

    def max_row(p, r):
        for cch in range(C // 16):
            sl = pl.ds(cch * 16, 16)
            acc = gbuf[p, 0, sl]
            for k in range(1, KP):
                acc = jnp.maximum(acc, gbuf[p, k, sl])
            aggb[r, sl] = acc

    def step(r, p, prefetch=True):
        wait_a(p)
        if prefetch:
            start_a(r + 1, 1 - p)
        scan_row(p)
        start_g(p)
        wait_g(1 - p)
        max_row(1 - p, r - 1)

    start_a(0, 0)
    wait_a(0)
    start_a(1, 1)
    scan_row(0)
    start_g(0)

    @pl.loop(1, ROWS_PER - 1, step=2)
    def _(r):
        step(r, 1)
        step(r + 1, 0)

    step(ROWS_PER - 1, 1, prefetch=False)
    wait_g(1)
    max_row(1, ROWS_PER - 1)

    pltpu.sync_copy(aggb, out_hbm.at[pl.ds(base, ROWS_PER)])


def _sage_sc(A2, xpad):
    mesh = plsc.VectorSubcoreMesh(core_axis_name="c", subcore_axis_name="s")
    cp = pltpu.CompilerParams()
    if "needs_layout_passes" in pltpu.CompilerParams.__dataclass_fields__:
        cp = dataclasses.replace(cp, needs_layout_passes=False)
    kfn = functools.partial(
        pl.kernel,
        mesh=mesh,
        compiler_params=cp,
        out_type=jax.ShapeDtypeStruct((NP, C), jnp.float32),
        scratch_types=[
            pltpu.VMEM((2, N), jnp.float32),
            pltpu.VMEM((2, KP), jnp.int32),
            pltpu.VMEM((2, KP, C), jnp.float32),
            pltpu.VMEM((ROWS_PER, C), jnp.float32),
            pltpu.VMEM((16,), jnp.int32),
            pltpu.SemaphoreType.DMA,
            pltpu.SemaphoreType.DMA,
            pltpu.SemaphoreType.DMA,
            pltpu.SemaphoreType.DMA,
        ],
    )(_sage_body)
    return kfn(A2, xpad)


def _mlp_body(x_ref, a_ref, w1_ref, w2_ref, b_ref, o_ref):
    acc = jnp.dot(x_ref[...], w1_ref[...], preferred_element_type=jnp.float32)
    acc += jnp.dot(a_ref[...], w2_ref[...], preferred_element_type=jnp.float32)
    o_ref[...] = jnp.maximum(acc + b_ref[...], 0.0)


def _mlp(X2, agg, W, b):
    MB = 1000
    return pl.pallas_call(
        _mlp_body,
        grid=(N // MB,),
        in_specs=[
            pl.BlockSpec((MB, C), lambda i: (i, 0)),
            pl.BlockSpec((MB, C), lambda i: (i, 0)),
            pl.BlockSpec((C, C), lambda i: (0, 0)),
            pl.BlockSpec((C, C), lambda i: (0, 0)),
            pl.BlockSpec((1, C), lambda i: (0, 0)),
        ],
        out_specs=pl.BlockSpec((MB, C), lambda i: (i, 0)),
        out_shape=jax.ShapeDtypeStruct((N, C), jnp.float32),
    )(X2, agg, W[:C], W[C:], b[None])


def kernel(A, X, agg_weights, agg_bias):
    A2 = A[0]
    X2 = X[0]
    xpad = jnp.pad(X2, ((0, XPAD_ROWS - N), (0, 0)))
    agg = _sage_sc(A2, xpad)[:N]
    out = _mlp(X2, agg, agg_weights, agg_bias)
    return out[None]


# speedup vs baseline: 9.5829x; 1.4951x over previous
import dataclasses
import functools

import jax
import jax.numpy as jnp
from jax import lax
from jax.experimental import pallas as pl
from jax.experimental.pallas import tpu as pltpu
from jax.experimental.pallas import tpu_sc as plsc

N = 10000
C = 128
K = 25
KP = 32

NW = 32
NG = 39
ROWS_PER = 320
NP = NW * ROWS_PER
XPAD_ROWS = N + 8


def _sage_body(a_hbm, xpad_hbm, out_hbm, abuf, idxb, gbuf, aggb, clist,
               sa0, sa1, sg0, sg1):
    wid = lax.axis_index("s") * 2 + lax.axis_index("c")
    base = wid * ROWS_PER
    iota16 = lax.iota(jnp.int32, 16)
    nfill = jnp.full((16,), N, jnp.int32)
    zeros16 = jnp.zeros((16,), jnp.int32)
    sa = (sa0, sa1)
    sg = (sg0, sg1)

    def a_row(r):
        return jnp.minimum(base + r, N - 1)

    def start_a(r, p):
        pltpu.make_async_copy(a_hbm.at[a_row(r)], abuf.at[p],
                              sa[p]).start()

    def wait_a(p):
        pltpu.make_async_copy(a_hbm.at[0], abuf.at[p],
                              sa[p]).wait()

    def start_g(p):
        pltpu.make_async_copy(xpad_hbm.at[idxb.at[p]], gbuf.at[p],
                              sg[p]).start()

    def wait_g(p):
        pltpu.make_async_copy(xpad_hbm.at[idxb.at[p]], gbuf.at[p],
                              sg[p]).wait()

    lane_eq = [iota16 == t for t in range(16)]

    def scan_row(p):
        idxb[p, pl.ds(0, 16)] = nfill
        idxb[p, pl.ds(16, 16)] = nfill

        def compact_chunks(flags, g, gcnt):
            m2 = flags != 0
            pos = gcnt + plsc.cumsum(m2.astype(jnp.int32)) - 1
            sm = jnp.logical_and(m2, pos < KP)
            posc = jnp.minimum(pos, KP - 1)
            plsc.store_scatter(clist, [posc], iota16 + g * 16, mask=sm)
            return gcnt + plsc.all_reduce_population_count(m2)

        def group(g, gcnt):
            flags = zeros16
            for t in range(16):
                v = abuf[p, pl.ds(g * 256 + t * 16, 16)]
                nz = lax.shift_left(plsc.bitcast(v, jnp.int32), 1) != 0
                pc = plsc.all_reduce_population_count(nz)
                flags = jnp.where(lane_eq[t], pc, flags)
            return compact_chunks(flags, g, gcnt)

        gcnt = lax.fori_loop(0, NG, group, zeros16, unroll=False)
        vlast = abuf[p, pl.ds(NG * 256, 16)]
        nzl = lax.shift_left(plsc.bitcast(vlast, jnp.int32), 1) != 0
        pcl = plsc.all_reduce_population_count(nzl)
        gcnt = compact_chunks(jnp.where(lane_eq[0], pcl, zeros16), NG, gcnt)
        nchunks = jnp.minimum(jnp.max(gcnt), KP)

        def chunk(j, cnt):
            jv = jnp.full((16,), 0, jnp.int32) + j
            cid = plsc.load_gather(clist, [jv])
            cols = cid * 16 + iota16
            v = plsc.load_gather(abuf, [jnp.full((16,), p, jnp.int32), cols])
            m = lax.shift_left(plsc.bitcast(v, jnp.int32), 1) != 0
            pos = cnt + plsc.cumsum(m.astype(jnp.int32)) - 1
            sm = jnp.logical_and(m, pos < K)
            posc = jnp.minimum(pos, KP - 1)
            plsc.store_scatter(idxb.at[p], [posc], cols, mask=sm)
            return cnt + plsc.all_reduce_population_count(m)

        lax.fori_loop(0, nchunks, chunk, zeros16, unroll=False)

        idx0 = plsc.load_gather(idxb.at[p], [zeros16])
        hi = idxb[p, pl.ds(16, 16)]
        idxb[p, pl.ds(16, 16)] = jnp.where(iota16 >= K - 16, idx0, hi)

    def max_row(p, r):
        for cch in range(C // 16):
            sl = pl.ds(cch * 16, 16)
            acc = gbuf[p, 0, sl]
            for k in range(1, KP):
                acc = jnp.maximum(acc, gbuf[p, k, sl])
            aggb[r, sl] = acc

    def step(r, p, prefetch=True):
        wait_a(p)
        if prefetch:
            start_a(r + 1, 1 - p)
        scan_row(p)
        start_g(p)
        wait_g(1 - p)
        max_row(1 - p, r - 1)

    start_a(0, 0)
    wait_a(0)
    start_a(1, 1)
    scan_row(0)
    start_g(0)

    @pl.loop(1, ROWS_PER - 1, step=2)
    def _(r):
        step(r, 1)
        step(r + 1, 0)

    step(ROWS_PER - 1, 1, prefetch=False)
    wait_g(1)
    max_row(1, ROWS_PER - 1)

    pltpu.sync_copy(aggb, out_hbm.at[pl.ds(base, ROWS_PER)])


def _sage_sc(A2, xpad):
    mesh = plsc.VectorSubcoreMesh(core_axis_name="c", subcore_axis_name="s")
    cp = pltpu.CompilerParams()
    if "needs_layout_passes" in pltpu.CompilerParams.__dataclass_fields__:
        cp = dataclasses.replace(cp, needs_layout_passes=False)
    kfn = functools.partial(
        pl.kernel,
        mesh=mesh,
        compiler_params=cp,
        out_type=jax.ShapeDtypeStruct((NP, C), jnp.float32),
        scratch_types=[
            pltpu.VMEM((2, N), jnp.float32),
            pltpu.VMEM((2, KP), jnp.int32),
            pltpu.VMEM((2, KP, C), jnp.float32),
            pltpu.VMEM((ROWS_PER, C), jnp.float32),
            pltpu.VMEM((KP,), jnp.int32),
            pltpu.SemaphoreType.DMA,
            pltpu.SemaphoreType.DMA,
            pltpu.SemaphoreType.DMA,
            pltpu.SemaphoreType.DMA,
        ],
    )(_sage_body)
    return kfn(A2, xpad)


def _mlp_body(x_ref, a_ref, w1_ref, w2_ref, b_ref, o_ref):
    acc = jnp.dot(x_ref[...], w1_ref[...], preferred_element_type=jnp.float32)
    acc += jnp.dot(a_ref[...], w2_ref[...], preferred_element_type=jnp.float32)
    o_ref[...] = jnp.maximum(acc + b_ref[...], 0.0)


def _mlp(X2, agg, W, b):
    MB = 1000
    return pl.pallas_call(
        _mlp_body,
        grid=(N // MB,),
        in_specs=[
            pl.BlockSpec((MB, C), lambda i: (i, 0)),
            pl.BlockSpec((MB, C), lambda i: (i, 0)),
            pl.BlockSpec((C, C), lambda i: (0, 0)),
            pl.BlockSpec((C, C), lambda i: (0, 0)),
            pl.BlockSpec((1, C), lambda i: (0, 0)),
        ],
        out_specs=pl.BlockSpec((MB, C), lambda i: (i, 0)),
        out_shape=jax.ShapeDtypeStruct((N, C), jnp.float32),
    )(X2, agg, W[:C], W[C:], b[None])


def kernel(A, X, agg_weights, agg_bias):
    A2 = A[0]
    X2 = X[0]
    xpad = jnp.pad(X2, ((0, XPAD_ROWS - N), (0, 0)))
    agg = _sage_sc(A2, xpad)[:N]
    out = _mlp(X2, agg, agg_weights, agg_bias)
    return out[None]
